# baseline (speedup 1.0000x reference)
import jax
import jax.numpy as jnp
from jax.experimental import pallas as pl
from jax.experimental.pallas import tpu as pltpu

_NB = 8
_BM = 256
_S1K = 32
_FLAT = 4096


def _conv_kernel(p1_ref, w1_ref, w2_ref, b2_ref, o_ref, h_ref):
    nb = _NB
    r = jax.lax.dot_general(
        p1_ref[...], w1_ref[...],
        dimension_numbers=(((0,), (0,)), ((), ())),
        preferred_element_type=jnp.float32)
    m = jnp.maximum(jnp.maximum(r[:, 0:64], r[:, 64:128]),
                    jnp.maximum(r[:, 128:192], r[:, 192:256]))
    h1 = jnp.maximum(m, 0.0).reshape(nb, 8, 2, 8, 64)

    h_ref[...] = jnp.zeros_like(h_ref)
    h_ref[:, 0, 0:8, 1:9, :] = h1[:, :, 0]
    h_ref[:, 1, 0:8, 1:9, :] = h1[:, :, 1]

    parts = []
    for p in range(4):
        e, r0 = p % 2, p // 2
        parts.append(h_ref[:, e, r0:r0 + 8, 1:9, :])
        parts.append(h_ref[:, e, r0:r0 + 8, 0:8, 32:64])
        parts.append(h_ref[:, e, r0:r0 + 8, 2:10, 0:32])
    pat = jnp.concatenate(parts, axis=-1).reshape(nb * 64, 512)
    acc = jnp.dot(pat, w2_ref[...], preferred_element_type=jnp.float32)

    m2 = jnp.maximum(jnp.maximum(acc[:, 0:64], acc[:, 64:128]),
                     jnp.maximum(acc[:, 128:192], acc[:, 192:256]))
    o_ref[...] = jnp.maximum(m2 + b2_ref[...], 0.0)


def _conv_call(p1, w1, w2, b2, n_pad):
    return pl.pallas_call(
        _conv_kernel,
        out_shape=jax.ShapeDtypeStruct((n_pad * 64, 64), jnp.float32),
        grid=(n_pad // _NB,),
        in_specs=[
            pl.BlockSpec((_S1K, _NB * 128), lambda i: (0, i)),
            pl.BlockSpec((_S1K, 256), lambda i: (0, 0)),
            pl.BlockSpec((512, 256), lambda i: (0, 0)),
            pl.BlockSpec((1, 64), lambda i: (0, 0)),
        ],
        out_specs=pl.BlockSpec((_NB * 64, 64), lambda i: (i, 0)),
        scratch_shapes=[pltpu.VMEM((_NB, 2, 9, 10, 64), jnp.float32)],
        compiler_params=pltpu.CompilerParams(dimension_semantics=("parallel",)),
    )(p1, w1, w2, b2)


def _mlp_kernel(x_ref, w1_ref, b1_ref, w2_ref, b2_ref, o_ref):
    h = jnp.dot(x_ref[...], w1_ref[...], preferred_element_type=jnp.float32)
    h = jnp.maximum(h + b1_ref[...], 0.0)
    o_ref[...] = jnp.dot(h, w2_ref[...],
                         preferred_element_type=jnp.float32) + b2_ref[...]


def _mlp_call(x_flat, w1, b1, w2, b2):
    n = x_flat.shape[0]
    return pl.pallas_call(
        _mlp_kernel,
        out_shape=jax.ShapeDtypeStruct((n, 10), jnp.float32),
        grid=(n // _BM,),
        in_specs=[
            pl.BlockSpec((_BM, _FLAT), lambda i: (i, 0)),
            pl.BlockSpec((_FLAT, 128), lambda i: (0, 0)),
            pl.BlockSpec((1, 128), lambda i: (0, 0)),
            pl.BlockSpec((128, 10), lambda i: (0, 0)),
            pl.BlockSpec((1, 10), lambda i: (0, 0)),
        ],
        out_specs=pl.BlockSpec((_BM, 10), lambda i: (i, 0)),
        compiler_params=pltpu.CompilerParams(dimension_semantics=("parallel",)),
    )(x_flat, w1, b1, w2, b2)


def _build_patches(x):
    n = x.shape[0]
    xe = jnp.pad(x, ((0, 0), (1, 1), (1, 1)))
    taps = [jnp.pad(xe[:, a:a + 27:2, b:b + 25:4], ((0, 0), (1, 1), (0, 1)))
            for a in range(4) for b in range(6)]
    one = jnp.pad(jnp.ones((n, 14, 7), jnp.float32), ((0, 0), (1, 1), (0, 1)))
    p = jnp.stack(taps + [one], axis=0)
    p = jnp.pad(p, ((0, _S1K - 25), (0, 0), (0, 0), (0, 0)))
    return p.reshape(_S1K, n * 128)


def _build_w1(w1_packed):
    wk = w1_packed[0:9, 0:32]
    b1 = w1_packed[36:37, 0:32]
    z = jnp.zeros((32,), jnp.float32)
    rows = []
    for k in range(_S1K):
        a, b = divmod(k, 6)
        pieces = []
        for oy in range(2):
            for ox in range(2):
                for xsub in range(2):
                    if k == 24:
                        pieces.append(b1[0])
                        continue
                    ty, tx = a - oy, b - 2 * xsub - ox
                    if k < 24 and 0 <= ty < 3 and 0 <= tx < 3:
                        pieces.append(wk[ty * 3 + tx])
                    else:
                        pieces.append(z)
        rows.append(jnp.concatenate(pieces))
    return jnp.stack(rows)


def _build_w2(w2_stacked):
    zb = jnp.zeros((32, 64), jnp.float32)

    def blk(p, q):
        cols = []
        for dy in range(2):
            for dx in range(2):
                ty, tx = p - dy, q - dx
                if 0 <= ty < 3 and 0 <= tx < 3:
                    t = ty * 3 + tx
                    cols.append(w2_stacked[t * 32:(t + 1) * 32, :])
                else:
                    cols.append(zb)
        return jnp.concatenate(cols, axis=1)

    rows = []
    for p in range(4):
        for q in (1, 2, 0, 3):
            rows.append(blk(p, q))
    return jnp.concatenate(rows, axis=0)


def _build_wf(w1_fc):
    wf = w1_fc.reshape(16, 8, 64, 128)[0:14:2, 0:7]
    return jnp.pad(wf, ((0, 1), (0, 1), (0, 0), (0, 0))).reshape(_FLAT, 128)


def kernel(w1_packed, w2_stacked, b2, w1_fc, b1_fc, w2_fc, b2_fc, x_nchw):
    n = x_nchw.shape[0]
    n_pad = -(-n // _BM) * _BM
    x = x_nchw.reshape(n, 28, 28).astype(jnp.float32)
    if n_pad != n:
        x = jnp.pad(x, ((0, n_pad - n), (0, 0), (0, 0)))
    p1 = _build_patches(x)
    h2 = _conv_call(p1, _build_w1(w1_packed), _build_w2(w2_stacked),
                    b2, n_pad)
    out = _mlp_call(h2.reshape(n_pad, _FLAT), _build_wf(w1_fc),
                    b1_fc, w2_fc, b2_fc)
    return out[:n]


# speedup vs baseline: 1.4294x; 1.4294x over previous
import jax
import jax.numpy as jnp
from jax.experimental import pallas as pl
from jax.experimental.pallas import tpu as pltpu

_NB = 8
_BM = 256
_S1K = 32
_FLAT = 4096


def _conv_kernel(p1_ref, w1_ref, w2_ref, b2_ref, o_ref, h_ref):
    nb = _NB
    r = jnp.dot(p1_ref[...], w1_ref[...], preferred_element_type=jnp.float32)
    m = jnp.maximum(jnp.maximum(r[:, 0:64], r[:, 64:128]),
                    jnp.maximum(r[:, 128:192], r[:, 192:256]))
    h1 = jnp.maximum(m, 0.0).reshape(nb, 8, 2, 8, 64)

    h_ref[...] = jnp.zeros_like(h_ref)
    h_ref[:, 0, 0:8, 1:9, :] = h1[:, :, 0]
    h_ref[:, 1, 0:8, 1:9, :] = h1[:, :, 1]

    parts = []
    for p in range(4):
        e, r0 = p % 2, p // 2
        parts.append(h_ref[:, e, r0:r0 + 8, 1:9, :])
        parts.append(h_ref[:, e, r0:r0 + 8, 0:8, 32:64])
        parts.append(h_ref[:, e, r0:r0 + 8, 2:10, 0:32])
    pat = jnp.concatenate(parts, axis=-1).reshape(nb * 64, 512)
    acc = jnp.dot(pat, w2_ref[...], preferred_element_type=jnp.float32)

    m2 = jnp.maximum(jnp.maximum(acc[:, 0:64], acc[:, 64:128]),
                     jnp.maximum(acc[:, 128:192], acc[:, 192:256]))
    o_ref[...] = jnp.maximum(m2 + b2_ref[...], 0.0)


def _conv_call(p1, w1, w2, b2, n_pad):
    return pl.pallas_call(
        _conv_kernel,
        out_shape=jax.ShapeDtypeStruct((n_pad * 64, 64), jnp.float32),
        grid=(n_pad // _NB,),
        in_specs=[
            pl.BlockSpec((_NB * 128, _S1K), lambda i: (i, 0)),
            pl.BlockSpec((_S1K, 256), lambda i: (0, 0)),
            pl.BlockSpec((512, 256), lambda i: (0, 0)),
            pl.BlockSpec((1, 64), lambda i: (0, 0)),
        ],
        out_specs=pl.BlockSpec((_NB * 64, 64), lambda i: (i, 0)),
        scratch_shapes=[pltpu.VMEM((_NB, 2, 9, 10, 64), jnp.float32)],
        compiler_params=pltpu.CompilerParams(dimension_semantics=("parallel",)),
    )(p1, w1, w2, b2)


def _mlp_kernel(x_ref, w1_ref, b1_ref, w2_ref, b2_ref, o_ref):
    h = jnp.dot(x_ref[...], w1_ref[...], preferred_element_type=jnp.float32)
    h = jnp.maximum(h + b1_ref[...], 0.0)
    o_ref[...] = jnp.dot(h, w2_ref[...],
                         preferred_element_type=jnp.float32) + b2_ref[...]


def _mlp_call(x_flat, w1, b1, w2, b2):
    n = x_flat.shape[0]
    return pl.pallas_call(
        _mlp_kernel,
        out_shape=jax.ShapeDtypeStruct((n, 10), jnp.float32),
        grid=(n // _BM,),
        in_specs=[
            pl.BlockSpec((_BM, _FLAT), lambda i: (i, 0)),
            pl.BlockSpec((_FLAT, 128), lambda i: (0, 0)),
            pl.BlockSpec((1, 128), lambda i: (0, 0)),
            pl.BlockSpec((128, 10), lambda i: (0, 0)),
            pl.BlockSpec((1, 10), lambda i: (0, 0)),
        ],
        out_specs=pl.BlockSpec((_BM, 10), lambda i: (i, 0)),
        compiler_params=pltpu.CompilerParams(dimension_semantics=("parallel",)),
    )(x_flat, w1, b1, w2, b2)


def _build_patches(x):
    n = x.shape[0]
    xe = jnp.pad(x, ((0, 0), (1, 1), (1, 3)))
    ph = jnp.stack([xe[:, e::2, px::4] for e in (0, 1) for px in range(4)],
                   axis=1)
    taps = []
    for a in range(4):
        for b in range(6):
            pidx = (a % 2) * 4 + b % 4
            taps.append(ph[:, pidx, a // 2:a // 2 + 14, b // 4:b // 4 + 7])
    p = jnp.stack(taps, axis=-1)
    p = jnp.concatenate([p, jnp.ones((n, 14, 7, 1), jnp.float32)], axis=-1)
    p = jnp.pad(p, ((0, 0), (1, 1), (0, 1), (0, _S1K - 25)))
    return p.reshape(n * 128, _S1K)


def _build_w1(w1_packed):
    wk = w1_packed[0:9, 0:32]
    b1 = w1_packed[36:37, 0:32]
    z = jnp.zeros((32,), jnp.float32)
    rows = []
    for k in range(_S1K):
        a, b = divmod(k, 6)
        pieces = []
        for oy in range(2):
            for ox in range(2):
                for xsub in range(2):
                    if k == 24:
                        pieces.append(b1[0])
                        continue
                    ty, tx = a - oy, b - 2 * xsub - ox
                    if k < 24 and 0 <= ty < 3 and 0 <= tx < 3:
                        pieces.append(wk[ty * 3 + tx])
                    else:
                        pieces.append(z)
        rows.append(jnp.concatenate(pieces))
    return jnp.stack(rows)


def _build_w2(w2_stacked):
    zb = jnp.zeros((32, 64), jnp.float32)

    def blk(p, q):
        cols = []
        for dy in range(2):
            for dx in range(2):
                ty, tx = p - dy, q - dx
                if 0 <= ty < 3 and 0 <= tx < 3:
                    t = ty * 3 + tx
                    cols.append(w2_stacked[t * 32:(t + 1) * 32, :])
                else:
                    cols.append(zb)
        return jnp.concatenate(cols, axis=1)

    rows = []
    for p in range(4):
        for q in (1, 2, 0, 3):
            rows.append(blk(p, q))
    return jnp.concatenate(rows, axis=0)


def _build_wf(w1_fc):
    wf = w1_fc.reshape(16, 8, 64, 128)[0:14:2, 0:7]
    return jnp.pad(wf, ((0, 1), (0, 1), (0, 0), (0, 0))).reshape(_FLAT, 128)


def kernel(w1_packed, w2_stacked, b2, w1_fc, b1_fc, w2_fc, b2_fc, x_nchw):
    n = x_nchw.shape[0]
    n_pad = -(-n // _BM) * _BM
    x = x_nchw.reshape(n, 28, 28).astype(jnp.float32)
    if n_pad != n:
        x = jnp.pad(x, ((0, n_pad - n), (0, 0), (0, 0)))
    p1 = _build_patches(x)
    h2 = _conv_call(p1, _build_w1(w1_packed), _build_w2(w2_stacked),
                    b2, n_pad)
    out = _mlp_call(h2.reshape(n_pad, _FLAT), _build_wf(w1_fc),
                    b1_fc, w2_fc, b2_fc)
    return out[:n]


# speedup vs baseline: 1.7114x; 1.1973x over previous
import jax
import jax.numpy as jnp
from jax.experimental import pallas as pl
from jax.experimental.pallas import tpu as pltpu

_NB = 8
_BM = 256
_S1K = 32
_FLAT = 4096


def _conv_kernel(p1_ref, w1_ref, w2_ref, b2_ref, o_ref, h_ref):
    nb = _NB
    r = jnp.dot(p1_ref[...], w1_ref[...], preferred_element_type=jnp.float32)
    m = jnp.maximum(jnp.maximum(r[:, 0:64], r[:, 64:128]),
                    jnp.maximum(r[:, 128:192], r[:, 192:256]))
    h1 = jnp.maximum(m, 0.0).reshape(nb, 8, 2, 8, 64)

    h_ref[...] = jnp.zeros_like(h_ref)
    h_ref[:, 0, 0:8, 1:9, :] = h1[:, :, 0]
    h_ref[:, 1, 0:8, 1:9, :] = h1[:, :, 1]

    parts = []
    for p in range(4):
        e, r0 = p % 2, p // 2
        parts.append(h_ref[:, e, r0:r0 + 8, 1:9, :])
        parts.append(h_ref[:, e, r0:r0 + 8, 0:8, 32:64])
        parts.append(h_ref[:, e, r0:r0 + 8, 2:10, 0:32])
    pat = jnp.concatenate(parts, axis=-1).reshape(nb * 64, 512)
    acc = jnp.dot(pat, w2_ref[...], preferred_element_type=jnp.float32)

    m2 = jnp.maximum(jnp.maximum(acc[:, 0:64], acc[:, 64:128]),
                     jnp.maximum(acc[:, 128:192], acc[:, 192:256]))
    o_ref[...] = jnp.maximum(m2 + b2_ref[...], 0.0).astype(jnp.bfloat16)


def _conv_call(p1, w1, w2, b2, n_pad):
    return pl.pallas_call(
        _conv_kernel,
        out_shape=jax.ShapeDtypeStruct((n_pad * 64, 64), jnp.bfloat16),
        grid=(n_pad // _NB,),
        in_specs=[
            pl.BlockSpec((_NB * 128, _S1K), lambda i: (i, 0)),
            pl.BlockSpec((_S1K, 256), lambda i: (0, 0)),
            pl.BlockSpec((512, 256), lambda i: (0, 0)),
            pl.BlockSpec((1, 64), lambda i: (0, 0)),
        ],
        out_specs=pl.BlockSpec((_NB * 64, 64), lambda i: (i, 0)),
        scratch_shapes=[pltpu.VMEM((_NB, 2, 9, 10, 64), jnp.float32)],
        compiler_params=pltpu.CompilerParams(dimension_semantics=("parallel",)),
    )(p1, w1, w2, b2)


def _mlp_kernel(x_ref, w1_ref, b1_ref, w2_ref, b2_ref, o_ref):
    h = jnp.dot(x_ref[...], w1_ref[...], preferred_element_type=jnp.float32)
    h = jnp.maximum(h + b1_ref[...], 0.0)
    o_ref[...] = jnp.dot(h, w2_ref[...],
                         preferred_element_type=jnp.float32) + b2_ref[...]


def _mlp_call(x_flat, w1, b1, w2, b2):
    n = x_flat.shape[0]
    return pl.pallas_call(
        _mlp_kernel,
        out_shape=jax.ShapeDtypeStruct((n, 10), jnp.float32),
        grid=(n // _BM,),
        in_specs=[
            pl.BlockSpec((_BM, _FLAT), lambda i: (i, 0)),
            pl.BlockSpec((_FLAT, 128), lambda i: (0, 0)),
            pl.BlockSpec((1, 128), lambda i: (0, 0)),
            pl.BlockSpec((128, 10), lambda i: (0, 0)),
            pl.BlockSpec((1, 10), lambda i: (0, 0)),
        ],
        out_specs=pl.BlockSpec((_BM, 10), lambda i: (i, 0)),
        compiler_params=pltpu.CompilerParams(dimension_semantics=("parallel",)),
    )(x_flat, w1, b1, w2, b2)


def _build_patches(x):
    n = x.shape[0]
    xe = jnp.pad(x.astype(jnp.bfloat16), ((0, 0), (1, 1), (1, 3)))
    ph = jnp.stack([xe[:, e::2, px::4] for e in (0, 1) for px in range(4)],
                   axis=1)
    taps = []
    for a in range(4):
        for b in range(6):
            pidx = (a % 2) * 4 + b % 4
            taps.append(ph[:, pidx, a // 2:a // 2 + 14, b // 4:b // 4 + 7])
    p = jnp.stack(taps, axis=-1)
    p = jnp.concatenate([p, jnp.ones((n, 14, 7, 1), jnp.bfloat16)], axis=-1)
    p = jnp.pad(p, ((0, 0), (1, 1), (0, 1), (0, _S1K - 25)))
    return p.reshape(n * 128, _S1K)


def _build_w1(w1_packed):
    wk = w1_packed[0:9, 0:32]
    b1 = w1_packed[36:37, 0:32]
    z = jnp.zeros((32,), jnp.float32)
    rows = []
    for k in range(_S1K):
        a, b = divmod(k, 6)
        pieces = []
        for oy in range(2):
            for ox in range(2):
                for xsub in range(2):
                    if k == 24:
                        pieces.append(b1[0])
                        continue
                    ty, tx = a - oy, b - 2 * xsub - ox
                    if k < 24 and 0 <= ty < 3 and 0 <= tx < 3:
                        pieces.append(wk[ty * 3 + tx])
                    else:
                        pieces.append(z)
        rows.append(jnp.concatenate(pieces))
    return jnp.stack(rows)


def _build_w2(w2_stacked):
    zb = jnp.zeros((32, 64), jnp.float32)

    def blk(p, q):
        cols = []
        for dy in range(2):
            for dx in range(2):
                ty, tx = p - dy, q - dx
                if 0 <= ty < 3 and 0 <= tx < 3:
                    t = ty * 3 + tx
                    cols.append(w2_stacked[t * 32:(t + 1) * 32, :])
                else:
                    cols.append(zb)
        return jnp.concatenate(cols, axis=1)

    rows = []
    for p in range(4):
        for q in (1, 2, 0, 3):
            rows.append(blk(p, q))
    return jnp.concatenate(rows, axis=0)


def _build_wf(w1_fc):
    wf = w1_fc.reshape(16, 8, 64, 128)[0:14:2, 0:7]
    return jnp.pad(wf, ((0, 1), (0, 1), (0, 0), (0, 0))).reshape(_FLAT, 128)


def kernel(w1_packed, w2_stacked, b2, w1_fc, b1_fc, w2_fc, b2_fc, x_nchw):
    n = x_nchw.shape[0]
    n_pad = -(-n // _BM) * _BM
    x = x_nchw.reshape(n, 28, 28).astype(jnp.float32)
    if n_pad != n:
        x = jnp.pad(x, ((0, n_pad - n), (0, 0), (0, 0)))
    p1 = _build_patches(x)
    h2 = _conv_call(p1, _build_w1(w1_packed).astype(jnp.bfloat16),
                    _build_w2(w2_stacked), b2, n_pad)
    out = _mlp_call(h2.reshape(n_pad, _FLAT),
                    _build_wf(w1_fc).astype(jnp.bfloat16),
                    b1_fc, w2_fc, b2_fc)
    return out[:n]
